# baseline (speedup 1.0000x reference)
import functools

import jax
import jax.numpy as jnp
from jax import lax
from jax.experimental import pallas as pl
from jax.experimental.pallas import tpu as pltpu
from jax.experimental.pallas import tpu_sc as plsc

RB = 16
NW = 32
_NC = 2


def _make_meta_body(mt, e_num):
    def body(idc_ref, idr_ref, meta_ref, dst_ref):
        idc = idc_ref[...]
        idr = idr_ref[...]
        e_col = lax.broadcasted_iota(jnp.int32, (e_num, mt), 0)
        eq_e = (e_col == idr).astype(jnp.float32)
        counts = jnp.sum(eq_e, axis=1, keepdims=True)
        nblk = jnp.ceil(counts * (1.0 / RB))
        tril_e = (lax.broadcasted_iota(jnp.int32, (e_num, e_num), 0) >
                  lax.broadcasted_iota(jnp.int32, (e_num, e_num), 1)
                  ).astype(jnp.float32)
        astart = RB * lax.dot_general(
            tril_e, nblk, (((1,), (0,)), ((), ())),
            preferred_element_type=jnp.float32)
        tril_t = (lax.broadcasted_iota(jnp.int32, (mt, mt), 0) >
                  lax.broadcasted_iota(jnp.int32, (mt, mt), 1)
                  ).astype(jnp.float32)
        eq_t = (idc == idr).astype(jnp.float32)
        rank = jnp.sum(eq_t * tril_t, axis=1, keepdims=True)
        oh = (idc == lax.broadcasted_iota(jnp.int32, (mt, e_num), 1)
              ).astype(jnp.float32)
        a_tok = lax.dot_general(oh, astart, (((1,), (0,)), ((), ())),
                                preferred_element_type=jnp.float32)
        dst_ref[...] = (a_tok + rank + 0.5).astype(jnp.int32)
        meta_ref[...] = (jnp.concatenate([astart, nblk], axis=0)
                         + 0.5).astype(jnp.int32)
    return body


def _make_tc_body(e_num, epg):
    def body(meta_ref, x_ref, w1_ref, w2_ref, wrow_ref, out_ref):
        p = pl.program_id(0)
        for j in range(epg):
            e = epg * p + j
            a = meta_ref[e]
            nblk = meta_ref[e_num + e]

            @pl.when(nblk > 0)
            def _():
                def blk(i, _):
                    r0 = pl.multiple_of(a + i * RB, RB)
                    x = x_ref[pl.ds(r0, RB), :]
                    h = lax.dot_general(x, w1_ref[j],
                                        (((1,), (1,)), ((), ())),
                                        preferred_element_type=jnp.float32)
                    dff = h.shape[1] // 2
                    g = h[:, :dff]
                    act = (g / (1.0 + jnp.exp(-g))) * h[:, dff:]
                    y = lax.dot_general(act, w2_ref[j],
                                        (((1,), (1,)), ((), ())),
                                        preferred_element_type=jnp.float32)
                    out_ref[pl.ds(r0, RB), :] = (
                        y * wrow_ref[pl.ds(r0, RB), 0:1])
                    return 0

                lax.fori_loop(0, nblk, blk, 0)

    return body


def _make_sc_pack(m, d, p_rows):
    assert m % NW == 0
    bpw = m // NW
    mesh = plsc.VectorSubcoreMesh(core_axis_name="c", subcore_axis_name="s")

    @functools.partial(
        pl.kernel, mesh=mesh,
        out_type=[jax.ShapeDtypeStruct((p_rows, d), jnp.float32),
                  jax.ShapeDtypeStruct((p_rows, 128), jnp.float32)],
        scratch_types=[
            pltpu.VMEM((bpw, d), jnp.float32),
            pltpu.VMEM((bpw, 128), jnp.float32),
            pltpu.VMEM((bpw,), jnp.int32),
            pltpu.SemaphoreType.DMA,
        ],
    )
    def pack(x_hbm, dst_hbm, wpad_hbm, xp_hbm, wp_hbm,
             rows_v, wrows_v, idx_v, sem):
        wid = lax.axis_index("s") * _NC + lax.axis_index("c")
        base = wid * bpw
        pltpu.sync_copy(x_hbm.at[pl.ds(base, bpw)], rows_v)
        pltpu.sync_copy(wpad_hbm.at[pl.ds(base, bpw)], wrows_v)
        pltpu.sync_copy(dst_hbm.at[wid], idx_v)
        h1 = pltpu.async_copy(rows_v, xp_hbm.at[idx_v], sem)
        h2 = pltpu.async_copy(wrows_v, wp_hbm.at[idx_v], sem)
        h1.wait()
        h2.wait()

    return pack


def _make_sc_gather(d, b_rows, n_streams):
    assert d % 16 == 0 and b_rows % (8 * NW) == 0
    bpw = b_rows // NW
    assert bpw % n_streams == 0 and (bpw // n_streams) % 8 == 0
    seg = bpw // n_streams
    mesh = plsc.VectorSubcoreMesh(core_axis_name="c", subcore_axis_name="s")

    @functools.partial(
        pl.kernel, mesh=mesh,
        out_type=jax.ShapeDtypeStruct((b_rows, d), jnp.float32),
        scratch_types=[
            pltpu.VMEM((bpw,), jnp.int32),
            pltpu.VMEM((bpw, d), jnp.float32),
            pltpu.SemaphoreType.DMA,
        ],
    )
    def gather(table_hbm, idx_hbm, out_hbm, idx_v, rows_v, sem):
        wid = lax.axis_index("s") * _NC + lax.axis_index("c")
        base = wid * bpw
        pltpu.sync_copy(idx_hbm.at[pl.ds(base, bpw)], idx_v)
        handles = [
            pltpu.async_copy(
                table_hbm.at[idx_v.at[pl.ds(t * seg, seg)]],
                rows_v.at[pl.ds(t * seg, seg)], sem)
            for t in range(n_streams)
        ]
        for h in handles:
            h.wait()
        pltpu.sync_copy(rows_v, out_hbm.at[pl.ds(base, bpw)])

    return gather


def kernel(hidden_states, w1, w2, topk_weights, topk_ids):
    m, k_dim = hidden_states.shape
    e_num, n_dim, _ = w1.shape
    dff = n_dim // 2
    topk = topk_ids.shape[1]
    mt = m * topk

    p_rows = ((mt + e_num * (RB - 1) + 8 * NW - 1) // (8 * NW)) * (8 * NW)

    flat_ids = topk_ids.reshape(-1).astype(jnp.int32)
    flat_w = topk_weights.reshape(-1)

    meta2d, dst2d = pl.pallas_call(
        _make_meta_body(mt, e_num),
        grid=(1,),
        in_specs=[
            pl.BlockSpec((mt, 1), lambda i: (0, 0)),
            pl.BlockSpec((1, mt), lambda i: (0, 0)),
        ],
        out_specs=[
            pl.BlockSpec((2 * e_num, 1), lambda i: (0, 0)),
            pl.BlockSpec((mt, 1), lambda i: (0, 0)),
        ],
        out_shape=[
            jax.ShapeDtypeStruct((2 * e_num, 1), jnp.int32),
            jax.ShapeDtypeStruct((mt, 1), jnp.int32),
        ],
    )(flat_ids.reshape(mt, 1), flat_ids.reshape(1, mt))
    meta = meta2d.reshape(2 * e_num)
    dst = dst2d.reshape(mt)
    wpad = jnp.broadcast_to(flat_w[:, None], (mt, 128))

    x_padded, wrow_padded = _make_sc_pack(mt, k_dim, p_rows)(
        hidden_states, dst.reshape(NW, mt // NW), wpad)

    epg = 2
    assert e_num % epg == 0
    grid_spec = pltpu.PrefetchScalarGridSpec(
        num_scalar_prefetch=1,
        grid=(e_num // epg,),
        in_specs=[
            pl.BlockSpec((p_rows, k_dim), lambda e, mr: (0, 0)),
            pl.BlockSpec((epg, n_dim, k_dim), lambda e, mr: (e, 0, 0)),
            pl.BlockSpec((epg, k_dim, dff), lambda e, mr: (e, 0, 0)),
            pl.BlockSpec((p_rows, 128), lambda e, mr: (0, 0)),
        ],
        out_specs=pl.BlockSpec((p_rows, k_dim), lambda e, mr: (0, 0)),
    )
    y_padded = pl.pallas_call(
        _make_tc_body(e_num, epg),
        grid_spec=grid_spec,
        out_shape=jax.ShapeDtypeStruct((p_rows, k_dim), jnp.float32),
    )(meta, x_padded, w1, w2, wrow_padded)

    out = _make_sc_gather(k_dim, mt, 2)(y_padded, dst)
    return out
